# baseline (speedup 1.0000x reference)
import functools

import jax
import jax.numpy as jnp
from jax import lax
from jax.experimental import pallas as pl
from jax.experimental.pallas import tpu as pltpu
from jax.experimental.pallas import tpu_sc as plsc

VOCAB = 1000000
EMBED = 64
BATCH = 16384

NC = 2
NS = 16
NW = NC * NS
L = 16
NCOLT = (VOCAB + 1 + 127) // 128
CPW = (NCOLT + NW - 1) // NW
LCAP = 3 * BATCH
BCAP = 32
BSTR = BCAP + L
FB = 64
IDSC = 2048
DEPTH = 3
FLS = 2 * CPW
OUTROWS = 3 * BATCH + FB * NW


@functools.cache
def _gather3():
  mesh = plsc.VectorSubcoreMesh(core_axis_name="c", subcore_axis_name="s")
  out_t = jax.ShapeDtypeStruct((OUTROWS, 128), jnp.float32)

  @functools.partial(
      pl.kernel,
      out_type=out_t,
      mesh=mesh,
      compiler_params=pltpu.CompilerParams(use_tc_tiling_on_sc=True,
                                           needs_layout_passes=False),
      scratch_types=[
          pltpu.VMEM((IDSC,), jnp.int32),
          pltpu.VMEM((LCAP + L,), jnp.int32),
          pltpu.VMEM((CPW * BSTR + L,), jnp.int32),
          pltpu.VMEM((2, FB, 128), jnp.float32),
          pltpu.VMEM((2, FB), jnp.int32),
          pltpu.SMEM((2 * CPW + 2,), jnp.int32),
          pltpu.VMEM((DEPTH, EMBED, 128), jnp.float32),
          pltpu.VMEM((DEPTH, EMBED, 128), jnp.float32),
          pltpu.SemaphoreType.DMA((DEPTH,)),
          pltpu.SemaphoreType.DMA((DEPTH,)),
          pltpu.SemaphoreType.DMA((2,)),
      ],
  )
  def body(in_tok, ctx_tok, neg_tok, wt_in, wt_ctx, out,
           ids_v, list_v, bkt_v, rows_v, bidx_v, cnt_s, bufa_v, bufb_v,
           fsema, fsemb, flsem):
    wid = lax.axis_index("s") * NC + lax.axis_index("c")
    c0 = wid * CPW
    ncols = jnp.minimum(NCOLT - c0, CPW)
    toks = (in_tok, ctx_tok, neg_tok)
    iota = lax.iota(jnp.int32, L)
    lane0 = iota == 0
    trash0 = 3 * BATCH + wid * FB

    def scan_g(g, toks_ref, cnt0):
      def chunk(ci, cnt):
        pltpu.sync_copy(toks_ref.at[pl.ds(ci * IDSC, IDSC)], ids_v)

        def vec(j, cnt):
          v = ids_v[pl.ds(j * L, L)]
          col = lax.shift_right_logical(v, 7)
          lcol = col - c0
          m = (lcol >= 0) & (lcol < ncols)
          pack = ((ci * IDSC + j * L + iota)
                  | lax.shift_left(v & 127, 14)
                  | (g << 21)
                  | lax.shift_left(lcol, 23))
          pfx = plsc.cumsum(jnp.where(m, 1, 0).astype(jnp.int32))
          plsc.store_scatter(list_v, [cnt + pfx - 1], pack, mask=m)
          return cnt + plsc.all_reduce_population_count(m)[0]

        return lax.fori_loop(0, IDSC // L, vec, cnt, unroll=4)

      return lax.fori_loop(0, BATCH // IDSC, chunk, cnt0)

    cnt = jnp.int32(0)
    for g in range(3):
      cnt = scan_g(g, toks[g], cnt)

    def splat(x):
      return jnp.full((L,), x, jnp.int32)

    def fill_trash(fb):
      for q in range(FB // L):
        plsc.store_scatter(bidx_v, [splat(fb), iota + q * L],
                           trash0 + iota + q * L)

    def fire_fetch(lc):
      slot = lax.rem(lc, DEPTH)
      flags = cnt_s[CPW + lc]

      @pl.when((flags & 1) > 0)
      def _():
        pltpu.make_async_copy(
            wt_in.at[:, pl.ds((c0 + lc) * 128, 128)],
            bufa_v.at[slot], fsema.at[slot]).start()

      @pl.when((flags & 2) > 0)
      def _():
        pltpu.make_async_copy(
            wt_ctx.at[:, pl.ds((c0 + lc) * 128, 128)],
            bufb_v.at[slot], fsemb.at[slot]).start()

    def wait_fetch(lc, slot):
      flags = cnt_s[CPW + lc]

      @pl.when((flags & 1) > 0)
      def _():
        pltpu.make_async_copy(
            wt_in.at[:, pl.ds(0, 128)], bufa_v.at[0], fsema.at[slot]).wait()

      @pl.when((flags & 2) > 0)
      def _():
        pltpu.make_async_copy(
            wt_ctx.at[:, pl.ds(0, 128)], bufb_v.at[0], fsemb.at[slot]).wait()

    def start_flush(fb):
      cnt_s[FLS + fb] = cnt_s[FLS + fb] + 1
      pltpu.make_async_copy(
          rows_v.at[fb], out.at[bidx_v.at[fb]], flsem.at[fb]).start()

    def wait_flush(fb):
      pltpu.make_async_copy(
          rows_v.at[0], out.at[bidx_v.at[0]], flsem.at[fb]).wait()
      cnt_s[FLS + fb] = cnt_s[FLS + fb] - 1

    cnt_s[FLS] = 0
    cnt_s[FLS + 1] = 0
    fill_trash(0)
    fill_trash(1)

    def round_body(carry):
      cnt, fr = carry

      def zero(i, _):
        cnt_s[i] = 0
        return 0
      lax.fori_loop(0, 2 * CPW, zero, 0)

      def binchunk(j, w):
        pv = list_v[pl.ds(j * L, L)]
        for k in range(L):
          p = pv[k]
          active = (j * L + k) < cnt
          lc = jnp.minimum(lax.shift_right_logical(p, 23), CPW - 1)
          c = cnt_s[lc]
          ovf = c >= BCAP

          g = lax.shift_right_logical(p, 21) & 3

          @pl.when(active & ~ovf)
          def _(p=p, lc=lc, c=c, g=g):
            cnt_s[lc] = c + 1
            cnt_s[CPW + lc] = cnt_s[CPW + lc] | jnp.where(g == 0, 1, 2)
            plsc.store_scatter(bkt_v, [splat(lc * BSTR + c)], splat(p),
                               mask=lane0)

          @pl.when(active & ovf)
          def _(p=p, w=w):
            plsc.store_scatter(list_v, [splat(w)], splat(p), mask=lane0)

          w = w + jnp.where(active & ovf, 1, 0)
        return w

      w = lax.fori_loop(0, (cnt + L - 1) // L, binchunk, jnp.int32(0))

      for d in range(DEPTH):
        @pl.when(ncols > d)
        def _(d=d):
          fire_fetch(jnp.int32(d))

      def col_body(lc, fr):
        slot = lax.rem(lc, DEPTH)
        wait_fetch(lc, slot)

        def entry(e, fr):
          pe = bkt_v[pl.ds(lc * BSTR + e, L)][0]
          b = pe & 16383
          lane = lax.shift_right_logical(pe, 14) & 127
          g = lax.shift_right_logical(pe, 21) & 3
          fb = lax.shift_right_logical(fr, 6) & 1
          ri = fr & (FB - 1)

          @pl.when(ri == 0)
          def _():
            @pl.when(cnt_s[FLS + fb] > 0)
            def _():
              wait_flush(fb)
            fill_trash(fb)

          lanes = splat(lane)
          for q in range(EMBED // L):
            rows = iota + q * L
            va = plsc.load_gather(bufa_v, [splat(slot), rows, lanes])
            vb = plsc.load_gather(bufb_v, [splat(slot), rows, lanes])
            val = jnp.where(g == 0, va, vb)
            plsc.store_scatter(rows_v, [splat(fb), splat(ri), rows], val)
          plsc.store_scatter(bidx_v, [splat(fb), splat(ri)],
                             splat(g * BATCH + b), mask=lane0)

          @pl.when(ri == FB - 1)
          def _():
            start_flush(fb)
          return fr + 1

        fr = lax.fori_loop(0, cnt_s[lc], entry, fr)

        @pl.when(lc + DEPTH < ncols)
        def _():
          fire_fetch(lc + DEPTH)
        return fr

      fr = lax.fori_loop(0, ncols, col_body, fr)
      return w, fr

    def round_cond(carry):
      cnt, _ = carry
      return cnt > 0

    cnt, fr = lax.while_loop(round_cond, round_body, (cnt, jnp.int32(0)))

    fbp = lax.shift_right_logical(fr, 6) & 1

    @pl.when((fr & (FB - 1)) > 0)
    def _():
      @pl.when(cnt_s[FLS + fbp] > 0)
      def _():
        wait_flush(fbp)
      start_flush(fbp)

    for fb in range(2):
      @pl.when(cnt_s[FLS + fb] > 0)
      def _(fb=fb):
        wait_flush(fb)

  return body


def kernel(input_tokens, context_tokens, negative_context, W_in, W_ctx):
  f = _gather3()
  o = f(input_tokens.astype(jnp.int32),
        context_tokens.astype(jnp.int32),
        negative_context.astype(jnp.int32),
        W_in.T, W_ctx.T)
  return (o[:BATCH, :EMBED], o[BATCH:2 * BATCH, :EMBED],
          o[2 * BATCH:3 * BATCH, :EMBED])


# speedup vs baseline: 1.0505x; 1.0505x over previous
import functools

import jax
import jax.numpy as jnp
from jax import lax
from jax.experimental import pallas as pl
from jax.experimental.pallas import tpu as pltpu
from jax.experimental.pallas import tpu_sc as plsc

VOCAB = 1000000
EMBED = 64
BATCH = 16384

NC = 2
NS = 16
NW = NC * NS
L = 16
NCOLT = (VOCAB + 1 + 127) // 128
CPW = (NCOLT + NW - 1) // NW
LCAP = 3 * BATCH
BCAP = 32
BSTR = BCAP + L
FB = 64
IDSC = 2048
DEPTH = 3
FLS = 2 * CPW
OUTROWS = 3 * BATCH + FB * NW


@functools.cache
def _gather3():
  mesh = plsc.VectorSubcoreMesh(core_axis_name="c", subcore_axis_name="s")
  out_t = jax.ShapeDtypeStruct((OUTROWS, 128), jnp.float32)

  @functools.partial(
      pl.kernel,
      out_type=out_t,
      mesh=mesh,
      compiler_params=pltpu.CompilerParams(use_tc_tiling_on_sc=True,
                                           needs_layout_passes=False),
      scratch_types=[
          pltpu.VMEM((IDSC,), jnp.int32),
          pltpu.VMEM((LCAP + L,), jnp.int32),
          pltpu.VMEM((CPW * BSTR + L,), jnp.int32),
          pltpu.VMEM((2, FB, 128), jnp.float32),
          pltpu.VMEM((2, FB), jnp.int32),
          pltpu.SMEM((2 * CPW + 2,), jnp.int32),
          pltpu.VMEM((DEPTH, EMBED, 128), jnp.float32),
          pltpu.VMEM((DEPTH, EMBED, 128), jnp.float32),
          pltpu.SemaphoreType.DMA((DEPTH,)),
          pltpu.SemaphoreType.DMA((DEPTH,)),
          pltpu.SemaphoreType.DMA((2,)),
      ],
  )
  def body(in_tok, ctx_tok, neg_tok, wt_in, wt_ctx, out,
           ids_v, list_v, bkt_v, rows_v, bidx_v, cnt_s, bufa_v, bufb_v,
           fsema, fsemb, flsem):
    wid = lax.axis_index("s") * NC + lax.axis_index("c")
    c0 = wid * CPW
    ncols = jnp.minimum(NCOLT - c0, CPW)
    toks = (in_tok, ctx_tok, neg_tok)
    iota = lax.iota(jnp.int32, L)
    lane0 = iota == 0
    trash0 = 3 * BATCH + wid * FB

    def scan_g(g, toks_ref, cnt0):
      def chunk(ci, cnt):
        pltpu.sync_copy(toks_ref.at[pl.ds(ci * IDSC, IDSC)], ids_v)

        def vec(j, cnt):
          v = ids_v[pl.ds(j * L, L)]
          col = lax.shift_right_logical(v, 7)
          lcol = col - c0
          m = (lcol >= 0) & (lcol < ncols)
          pack = ((ci * IDSC + j * L + iota)
                  | lax.shift_left(v & 127, 14)
                  | (g << 21)
                  | lax.shift_left(lcol, 23))
          pfx = plsc.cumsum(jnp.where(m, 1, 0).astype(jnp.int32))
          plsc.store_scatter(list_v, [cnt + pfx - 1], pack, mask=m)
          return cnt + plsc.all_reduce_population_count(m)[0]

        return lax.fori_loop(0, IDSC // L, vec, cnt, unroll=4)

      return lax.fori_loop(0, BATCH // IDSC, chunk, cnt0)

    cnt = jnp.int32(0)
    for g in range(3):
      cnt = scan_g(g, toks[g], cnt)

    def splat(x):
      return jnp.full((L,), x, jnp.int32)

    def fill_trash(fb):
      for q in range(FB // L):
        plsc.store_scatter(bidx_v, [splat(fb), iota + q * L],
                           trash0 + iota + q * L)

    def fire_fetch(lc):
      slot = lax.rem(lc, DEPTH)
      flags = cnt_s[CPW + lc]

      @pl.when((flags & 1) > 0)
      def _():
        pltpu.make_async_copy(
            wt_in.at[:, pl.ds((c0 + lc) * 128, 128)],
            bufa_v.at[slot], fsema.at[slot]).start()

      @pl.when((flags & 2) > 0)
      def _():
        pltpu.make_async_copy(
            wt_ctx.at[:, pl.ds((c0 + lc) * 128, 128)],
            bufb_v.at[slot], fsemb.at[slot]).start()

    def wait_fetch(lc, slot):
      flags = cnt_s[CPW + lc]

      @pl.when((flags & 1) > 0)
      def _():
        pltpu.make_async_copy(
            wt_in.at[:, pl.ds(0, 128)], bufa_v.at[0], fsema.at[slot]).wait()

      @pl.when((flags & 2) > 0)
      def _():
        pltpu.make_async_copy(
            wt_ctx.at[:, pl.ds(0, 128)], bufb_v.at[0], fsemb.at[slot]).wait()

    def start_flush(fb):
      cnt_s[FLS + fb] = cnt_s[FLS + fb] + 1
      pltpu.make_async_copy(
          rows_v.at[fb], out.at[bidx_v.at[fb]], flsem.at[fb]).start()

    def wait_flush(fb):
      pltpu.make_async_copy(
          rows_v.at[0], out.at[bidx_v.at[0]], flsem.at[fb]).wait()
      cnt_s[FLS + fb] = cnt_s[FLS + fb] - 1

    cnt_s[FLS] = 0
    cnt_s[FLS + 1] = 0
    fill_trash(0)
    fill_trash(1)

    def round_body(carry):
      cnt, fr = carry

      def zero(i, _):
        cnt_s[i] = 0
        return 0
      lax.fori_loop(0, 2 * CPW, zero, 0)

      def binchunk(j, w):
        pv = list_v[pl.ds(j * L, L)]
        for k in range(L):
          p = pv[k]
          active = (j * L + k) < cnt
          lc = jnp.minimum(lax.shift_right_logical(p, 23), CPW - 1)
          c = cnt_s[lc]
          ovf = c >= BCAP

          g = lax.shift_right_logical(p, 21) & 3

          @pl.when(active & ~ovf)
          def _(p=p, lc=lc, c=c, g=g):
            cnt_s[lc] = c + 1
            cnt_s[CPW + lc] = cnt_s[CPW + lc] | jnp.where(g == 0, 1, 2)
            plsc.store_scatter(bkt_v, [splat(lc * BSTR + c)], splat(p),
                               mask=lane0)

          @pl.when(active & ovf)
          def _(p=p, w=w):
            plsc.store_scatter(list_v, [splat(w)], splat(p), mask=lane0)

          w = w + jnp.where(active & ovf, 1, 0)
        return w

      w = lax.fori_loop(0, (cnt + L - 1) // L, binchunk, jnp.int32(0))

      for d in range(DEPTH):
        @pl.when(ncols > d)
        def _(d=d):
          fire_fetch(jnp.int32(d))

      def col_body(lc, fr):
        slot = lax.rem(lc, DEPTH)
        wait_fetch(lc, slot)

        def entry(e, fr):
          pe = bkt_v[pl.ds(lc * BSTR + e, L)][0]
          b = pe & 16383
          lane = lax.shift_right_logical(pe, 14) & 127
          g = lax.shift_right_logical(pe, 21) & 3
          fb = lax.shift_right_logical(fr, 6) & 1
          ri = fr & (FB - 1)

          @pl.when(ri == 0)
          def _():
            @pl.when(cnt_s[FLS + fb] > 0)
            def _():
              wait_flush(fb)
            fill_trash(fb)

          lanes = splat(lane)

          @pl.when(g == 0)
          def _():
            for q in range(EMBED // L):
              rows = iota + q * L
              va = plsc.load_gather(bufa_v, [splat(slot), rows, lanes])
              plsc.store_scatter(rows_v, [splat(fb), splat(ri), rows], va)

          @pl.when(g != 0)
          def _():
            for q in range(EMBED // L):
              rows = iota + q * L
              vb = plsc.load_gather(bufb_v, [splat(slot), rows, lanes])
              plsc.store_scatter(rows_v, [splat(fb), splat(ri), rows], vb)
          plsc.store_scatter(bidx_v, [splat(fb), splat(ri)],
                             splat(g * BATCH + b), mask=lane0)

          @pl.when(ri == FB - 1)
          def _():
            start_flush(fb)
          return fr + 1

        fr = lax.fori_loop(0, cnt_s[lc], entry, fr)

        @pl.when(lc + DEPTH < ncols)
        def _():
          fire_fetch(lc + DEPTH)
        return fr

      fr = lax.fori_loop(0, ncols, col_body, fr)
      return w, fr

    def round_cond(carry):
      cnt, _ = carry
      return cnt > 0

    cnt, fr = lax.while_loop(round_cond, round_body, (cnt, jnp.int32(0)))

    fbp = lax.shift_right_logical(fr, 6) & 1

    @pl.when((fr & (FB - 1)) > 0)
    def _():
      @pl.when(cnt_s[FLS + fbp] > 0)
      def _():
        wait_flush(fbp)
      start_flush(fbp)

    for fb in range(2):
      @pl.when(cnt_s[FLS + fb] > 0)
      def _(fb=fb):
        wait_flush(fb)

  return body


def kernel(input_tokens, context_tokens, negative_context, W_in, W_ctx):
  f = _gather3()
  o = f(input_tokens.astype(jnp.int32),
        context_tokens.astype(jnp.int32),
        negative_context.astype(jnp.int32),
        W_in.T, W_ctx.T)
  return (o[:BATCH, :EMBED], o[BATCH:2 * BATCH, :EMBED],
          o[2 * BATCH:3 * BATCH, :EMBED])
